# baseline (speedup 1.0000x reference)
import jax
import jax.numpy as jnp
from jax.experimental import pallas as pl
from jax.experimental.pallas import tpu as pltpu

_ROWS = 64 * 32
_COLS = 32768
_BLOCK_ROWS = 64


def _ew_kernel(winc_ref, wthr_ref, a_ref, b_ref, x_ref, o_ref):
    w = winc_ref[0, 0] * a_ref[0]
    bv = wthr_ref[0, 0] * b_ref[0]
    xv = x_ref[...]
    y = jnp.where(xv < bv, 0.0, xv) * w
    o_ref[...] = jnp.where(y > 1.0, 1.0, y)


def kernel(x, w_inc, w_thr, a, b):
    x2 = x.reshape(_ROWS, _COLS)
    out = pl.pallas_call(
        _ew_kernel,
        grid=(_ROWS // _BLOCK_ROWS,),
        in_specs=[
            pl.BlockSpec(memory_space=pltpu.SMEM),
            pl.BlockSpec(memory_space=pltpu.SMEM),
            pl.BlockSpec(memory_space=pltpu.SMEM),
            pl.BlockSpec(memory_space=pltpu.SMEM),
            pl.BlockSpec((_BLOCK_ROWS, _COLS), lambda i: (i, 0)),
        ],
        out_specs=pl.BlockSpec((_BLOCK_ROWS, _COLS), lambda i: (i, 0)),
        out_shape=jax.ShapeDtypeStruct((_ROWS, _COLS), x.dtype),
    )(w_inc, w_thr, a, b, x2)
    return out.reshape(x.shape)
